# baseline (speedup 1.0000x reference)
import jax
import jax.numpy as jnp
from jax import lax
from jax.experimental import pallas as pl
from jax.experimental.pallas import tpu as pltpu

FUSED_VOCAB = 128
EDGE_TILE = 8192
NODE_PACK = 4
EDGE_PACK = 32
EDGE_SUB = 8


def _round_up(x, m):
    return ((x + m - 1) // m) * m


def kernel(emb_w, tag_w, per_w, grp_w, lin_w, lin_b, lin_e_w, lin_e_b,
           period_table, group_table, z, tag, rel_pos, edge_attr):
    n = z.shape[0]
    e = rel_pos.shape[0]
    n_elements = emb_w.shape[0]
    atom_dim = emb_w.shape[1]
    tag_dim = tag_w.shape[1]
    pg_dim = per_w.shape[1]
    hidden = lin_w.shape[1]
    rp_dim = rel_pos.shape[1]
    ea_dim = edge_attr.shape[1]

    def fused_kernel(z4_ref, t4_ref, c_ref, rp_ref, ea_ref,
                     wr_ref, we_ref, be_ref, h4_ref, e32_ref):
        rows = z4_ref.shape[0]
        lanes = lax.broadcasted_iota(jnp.int32, (rows, FUSED_VOCAB), 1)
        pieces = []
        for k in range(NODE_PACK):
            mh = ((lanes == z4_ref[:, k:k + 1])
                  | (lanes == t4_ref[:, k:k + 1] + n_elements))
            pieces.append(jnp.dot(mh.astype(jnp.float32), c_ref[...],
                                  preferred_element_type=jnp.float32))
        h4_ref[...] = jnp.concatenate(pieces, axis=1)

        ea_blk = ea_ref[...]
        w = EDGE_SUB * ea_dim
        parts = [jnp.dot(ea_blk[:, w * k:w * (k + 1)], we_ref[...],
                         preferred_element_type=jnp.float32)
                 for k in range(EDGE_PACK // EDGE_SUB)]
        e32_ref[...] = (jnp.concatenate(parts, axis=1)
                        + jnp.dot(rp_ref[...], wr_ref[...],
                                  preferred_element_type=jnp.float32)
                        + be_ref[...])

    emb_eff = jnp.dot(emb_w, lin_w[:atom_dim], preferred_element_type=jnp.float32)
    tag_eff = jnp.dot(tag_w, lin_w[atom_dim:atom_dim + tag_dim],
                      preferred_element_type=jnp.float32)
    per_eff = jnp.dot(per_w, lin_w[atom_dim + tag_dim:atom_dim + tag_dim + pg_dim],
                      preferred_element_type=jnp.float32)
    grp_eff = jnp.dot(grp_w, lin_w[atom_dim + tag_dim + pg_dim:],
                      preferred_element_type=jnp.float32)
    a_rows = (emb_eff + per_eff[period_table] + grp_eff[group_table]
              + lin_b.astype(jnp.float32))
    c = jnp.zeros((FUSED_VOCAB, hidden), jnp.float32)
    c = lax.dynamic_update_slice(c, a_rows, (0, 0))
    c = lax.dynamic_update_slice(c, tag_eff, (n_elements, 0))

    w3 = lin_e_w[:rp_dim].astype(jnp.float32)
    w16 = lin_e_w[rp_dim:].astype(jnp.float32)
    wr = jnp.kron(jnp.eye(EDGE_PACK, dtype=jnp.float32), w3)
    we = jnp.kron(jnp.eye(EDGE_SUB, dtype=jnp.float32), w16)
    b32 = jnp.tile(lin_e_b.astype(jnp.float32), (1, EDGE_PACK))

    e_pad = _round_up(max(e, 1), EDGE_TILE)
    g = e_pad // EDGE_TILE
    tn = _round_up(-(-max(n, 1) // g), 8 * NODE_PACK)
    n_pad = g * tn
    zc = z.astype(jnp.int32)
    tc = tag.astype(jnp.int32)
    rp = rel_pos.astype(jnp.float32)
    ea = edge_attr.astype(jnp.float32)
    if n_pad != n:
        zc = jnp.pad(zc, (0, n_pad - n))
        tc = jnp.pad(tc, (0, n_pad - n))
    if e_pad != e:
        rp = jnp.pad(rp, ((0, e_pad - e), (0, 0)))
        ea = jnp.pad(ea, ((0, e_pad - e), (0, 0)))

    z4 = zc.reshape(n_pad // NODE_PACK, NODE_PACK)
    t4 = tc.reshape(n_pad // NODE_PACK, NODE_PACK)
    rp32 = rp.reshape(e_pad // EDGE_PACK, EDGE_PACK * rp_dim)
    ea32 = ea.reshape(e_pad // EDGE_PACK, EDGE_PACK * ea_dim)

    tn4 = tn // NODE_PACK
    te32 = EDGE_TILE // EDGE_PACK

    h4, e32 = pl.pallas_call(
        fused_kernel,
        out_shape=(jax.ShapeDtypeStruct((n_pad // NODE_PACK, NODE_PACK * hidden),
                                        jnp.float32),
                   jax.ShapeDtypeStruct((e_pad // EDGE_PACK, EDGE_PACK * hidden),
                                        jnp.float32)),
        grid=(g,),
        in_specs=[
            pl.BlockSpec((tn4, NODE_PACK), lambda i: (i, 0)),
            pl.BlockSpec((tn4, NODE_PACK), lambda i: (i, 0)),
            pl.BlockSpec((FUSED_VOCAB, hidden), lambda i: (0, 0)),
            pl.BlockSpec((te32, EDGE_PACK * rp_dim), lambda i: (i, 0)),
            pl.BlockSpec((te32, EDGE_PACK * ea_dim), lambda i: (i, 0)),
            pl.BlockSpec((EDGE_PACK * rp_dim, EDGE_PACK * hidden),
                         lambda i: (0, 0)),
            pl.BlockSpec((EDGE_SUB * ea_dim, EDGE_SUB * hidden),
                         lambda i: (0, 0)),
            pl.BlockSpec((1, EDGE_PACK * hidden), lambda i: (0, 0)),
        ],
        out_specs=(pl.BlockSpec((tn4, NODE_PACK * hidden), lambda i: (i, 0)),
                   pl.BlockSpec((te32, EDGE_PACK * hidden), lambda i: (i, 0))),
        compiler_params=pltpu.CompilerParams(
            dimension_semantics=("parallel",)),
    )(z4, t4, c, rp32, ea32, wr, we, b32)

    h = h4.reshape(n_pad, hidden)
    e_out = e32.reshape(e_pad, hidden)
    if n_pad != n:
        h = h[:n]
    if e_pad != e:
        e_out = e_out[:e]
    return h, e_out
